# baseline (speedup 1.0000x reference)
import functools

import jax
import jax.numpy as jnp
from jax import lax
from jax.experimental import pallas as pl
from jax.experimental.pallas import tpu as pltpu
from jax.experimental.pallas import tpu_sc as plsc

_NC, _NS = 2, 16
_NW = _NC * _NS
_NB = 8


@functools.lru_cache(maxsize=None)
def _build_gather(B: int, V: int, D: int, chunk: int):
    b_per_w = B // _NW
    n = b_per_w // chunk
    G = n // _NB
    assert b_per_w % chunk == 0 and n % _NB == 0 and G >= 2 and chunk % 8 == 0

    mesh = plsc.VectorSubcoreMesh(
        core_axis_name="c", subcore_axis_name="s",
        num_cores=_NC, num_subcores=_NS)

    @functools.partial(
        pl.kernel,
        out_type=jax.ShapeDtypeStruct((B, D), jnp.float32),
        mesh=mesh,
        scratch_types=[
            pltpu.VMEM((b_per_w,), jnp.int32),
            pltpu.VMEM((_NB, chunk, D), jnp.float32),
            pltpu.VMEM_SHARED((V, D), jnp.float32),
            pltpu.VMEM((1, D), jnp.float32),
        ] + [pltpu.SemaphoreType.DMA] * (2 * _NB),
    )
    def gather_kernel(table_hbm, idx_hbm, out_hbm, idx_v, rows_v, tbl_s, zrow, *sems):
        lag = 2
        gsem, osem = sems[:_NB], sems[_NB:]
        wid = lax.axis_index("s") * _NC + lax.axis_index("c")
        base = wid * b_per_w
        pltpu.sync_copy(idx_hbm.at[pl.ds(base, b_per_w)], idx_v)

        piece = 64
        tail = (V - 1) - (_NS - 1) * piece
        assert (_NS - 1) * piece + tail == V - 1 and tail % 8 == 0
        assert piece * D * 4 <= _NB * chunk * D * 4
        buf = rows_v.reshape(_NB * chunk, D)
        s_id = lax.axis_index("s")
        p0 = s_id * piece

        @pl.when(s_id < _NS - 1)
        def _stage_main():
            pltpu.sync_copy(table_hbm.at[pl.ds(p0, piece)], buf.at[pl.ds(0, piece)])
            pltpu.sync_copy(buf.at[pl.ds(0, piece)], tbl_s.at[pl.ds(p0, piece)])

        @pl.when(s_id == _NS - 1)
        def _stage_tail():
            pltpu.sync_copy(table_hbm.at[pl.ds(p0, tail)], buf.at[pl.ds(0, tail)])
            pltpu.sync_copy(buf.at[pl.ds(0, tail)], tbl_s.at[pl.ds(p0, tail)])
            zero = jnp.zeros((16,), jnp.float32)
            for k in range(D // 16):
                zrow.at[0][pl.ds(k * 16, 16)] = zero
            pltpu.sync_copy(zrow, tbl_s.at[pl.ds(V - 1, 1)])

        plsc.subcore_barrier()

        def gather_desc(i, b):
            return pltpu.make_async_copy(
                tbl_s.at[idx_v.at[pl.ds(i * chunk, chunk)]],
                rows_v.at[b], gsem[b])

        def out_desc(i, b):
            return pltpu.make_async_copy(
                rows_v.at[b], out_hbm.at[pl.ds(base + i * chunk, chunk)],
                osem[b])

        def step(i, b, wait_out, fire_next):
            b2 = (b + lag) % _NB
            gather_desc(i, b).wait()
            out_desc(i, b).start()
            if wait_out:
                out_desc(i - (_NB - lag), b2).wait()
            if fire_next:
                gather_desc(i + lag, b2).start()

        for b in range(lag):
            gather_desc(b, b).start()

        for b in range(_NB):
            step(b, b, b >= _NB - lag, True)

        @pl.loop(1, G - 1)
        def _steady(g):
            i0 = g * _NB
            for b in range(_NB):
                step(i0 + b, b, True, True)

        i0 = n - _NB
        for b in range(_NB):
            step(i0 + b, b, True, b < _NB - lag)

        for b in range(lag, _NB):
            out_desc(n - _NB + b, b).wait()

    return gather_kernel


def kernel(indices, table):
    batch, hist = indices.shape
    V, D = table.shape
    idx = indices.T.reshape(batch * hist).astype(jnp.int32)
    out = _build_gather(batch * hist, V, D, 80)(table, idx)
    return out.reshape(hist, batch, D).transpose(1, 0, 2)


# speedup vs baseline: 1.0178x; 1.0178x over previous
import functools

import jax
import jax.numpy as jnp
from jax import lax
from jax.experimental import pallas as pl
from jax.experimental.pallas import tpu as pltpu
from jax.experimental.pallas import tpu_sc as plsc

_NC, _NS = 2, 16
_NW = _NC * _NS
_NB = 8


@functools.lru_cache(maxsize=None)
def _build_gather(B: int, V: int, D: int, chunk: int):
    b_per_w = B // _NW
    n = b_per_w // chunk
    G = n // _NB
    assert b_per_w % chunk == 0 and n % _NB == 0 and G >= 2 and chunk % 8 == 0

    mesh = plsc.VectorSubcoreMesh(
        core_axis_name="c", subcore_axis_name="s",
        num_cores=_NC, num_subcores=_NS)

    @functools.partial(
        pl.kernel,
        out_type=jax.ShapeDtypeStruct((B, D), jnp.float32),
        mesh=mesh,
        scratch_types=[
            pltpu.VMEM((b_per_w,), jnp.int32),
            pltpu.VMEM((_NB, chunk, D), jnp.float32),
            pltpu.VMEM_SHARED((V, D), jnp.float32),
            pltpu.VMEM((1, D), jnp.float32),
        ] + [pltpu.SemaphoreType.DMA] * (2 * _NB),
    )
    def gather_kernel(table_hbm, idx_hbm, out_hbm, idx_v, rows_v, tbl_s, zrow, *sems):
        lag = _NB // 2
        gsem, osem = sems[:_NB], sems[_NB:]
        wid = lax.axis_index("s") * _NC + lax.axis_index("c")
        base = wid * b_per_w
        pltpu.sync_copy(idx_hbm.at[pl.ds(base, b_per_w)], idx_v)

        piece = 64
        tail = (V - 1) - (_NS - 1) * piece
        assert (_NS - 1) * piece + tail == V - 1 and tail % 8 == 0
        assert piece * D * 4 <= _NB * chunk * D * 4
        buf = rows_v.reshape(_NB * chunk, D)
        s_id = lax.axis_index("s")
        p0 = s_id * piece

        @pl.when(s_id < _NS - 1)
        def _stage_main():
            pltpu.sync_copy(table_hbm.at[pl.ds(p0, piece)], buf.at[pl.ds(0, piece)])
            pltpu.sync_copy(buf.at[pl.ds(0, piece)], tbl_s.at[pl.ds(p0, piece)])

        @pl.when(s_id == _NS - 1)
        def _stage_tail():
            pltpu.sync_copy(table_hbm.at[pl.ds(p0, tail)], buf.at[pl.ds(0, tail)])
            pltpu.sync_copy(buf.at[pl.ds(0, tail)], tbl_s.at[pl.ds(p0, tail)])
            zero = jnp.zeros((16,), jnp.float32)
            for k in range(D // 16):
                zrow.at[0][pl.ds(k * 16, 16)] = zero
            pltpu.sync_copy(zrow, tbl_s.at[pl.ds(V - 1, 1)])

        plsc.subcore_barrier()

        def gather_desc(i, b):
            return pltpu.make_async_copy(
                tbl_s.at[idx_v.at[pl.ds(i * chunk, chunk)]],
                rows_v.at[b], gsem[b])

        def out_desc(i, b):
            return pltpu.make_async_copy(
                rows_v.at[b], out_hbm.at[pl.ds(base + i * chunk, chunk)],
                osem[b])

        def step(i, b, wait_out, fire_next):
            b2 = (b + lag) % _NB
            gather_desc(i, b).wait()
            out_desc(i, b).start()
            if wait_out:
                out_desc(i - (_NB - lag), b2).wait()
            if fire_next:
                gather_desc(i + lag, b2).start()

        for b in range(lag):
            gather_desc(b, b).start()

        for b in range(_NB):
            step(b, b, b >= _NB - lag, True)

        @pl.loop(1, G - 1)
        def _steady(g):
            i0 = g * _NB
            for b in range(_NB):
                step(i0 + b, b, True, True)

        i0 = n - _NB
        for b in range(_NB):
            step(i0 + b, b, True, b < _NB - lag)

        for b in range(lag, _NB):
            out_desc(n - _NB + b, b).wait()

    return gather_kernel


def kernel(indices, table):
    batch, hist = indices.shape
    V, D = table.shape
    idx = indices.T.reshape(batch * hist).astype(jnp.int32)
    out = _build_gather(batch * hist, V, D, 80)(table, idx)
    return out.reshape(hist, batch, D).transpose(1, 0, 2)
